# baseline (speedup 1.0000x reference)
import jax
import jax.numpy as jnp
from jax.experimental import pallas as pl
from jax.experimental.pallas import tpu as pltpu

UTT_LEN = 20
N = 4096
VOCAB1 = 129


def _onehot_fill(src_ref, o_ref, zbuf, dbuf, zsems, dsems):
    zbuf[...] = jnp.zeros((N, 1), jnp.float32)
    zcopies = []
    for i in range(UTT_LEN):
        c = pltpu.make_async_copy(
            zbuf, o_ref.at[jnp.int32(i), :, 128:129], zsems.at[jnp.int32(i)]
        )
        c.start()
        zcopies.append(c)
    lane = jax.lax.broadcasted_iota(jnp.int32, (N, 128), 1)
    dcopies = [None, None]
    for i in range(UTT_LEN):
        b = i % 2
        if dcopies[b] is not None:
            dcopies[b].wait()
        s = src_ref[i, 0, :]
        dbuf[b] = jnp.where(lane == 0, s[:, None], jnp.float32(0.0))
        c = pltpu.make_async_copy(
            dbuf.at[jnp.int32(b)], o_ref.at[jnp.int32(i), :, 0:128],
            dsems.at[jnp.int32(b)]
        )
        c.start()
        dcopies[b] = c
    for c in dcopies:
        c.wait()
    for c in zcopies:
        c.wait()


def kernel(meanings, src):
    del meanings
    src3 = src.astype(jnp.float32).reshape(UTT_LEN, 1, N)
    return pl.pallas_call(
        _onehot_fill,
        in_specs=[pl.BlockSpec(memory_space=pltpu.MemorySpace.VMEM)],
        out_specs=pl.BlockSpec(memory_space=pltpu.MemorySpace.HBM),
        out_shape=jax.ShapeDtypeStruct((UTT_LEN, N, VOCAB1), jnp.float32),
        scratch_shapes=[
            pltpu.MemorySpace.VMEM((N, 1), jnp.float32),
            pltpu.MemorySpace.VMEM((2, N, 128), jnp.float32),
            pltpu.SemaphoreType.DMA((UTT_LEN,)),
            pltpu.SemaphoreType.DMA((2,)),
        ],
    )(src3)
